# initial kernel scaffold (unmeasured)
import jax
import jax.numpy as jnp
from jax import lax
from jax.experimental import pallas as pl
from jax.experimental.pallas import tpu as pltpu


def kernel(
    x,
):
    def body(*refs):
        pass

    out_shape = jax.ShapeDtypeStruct(..., jnp.float32)
    return pl.pallas_call(body, out_shape=out_shape)(...)


# baseline (device time: 191328 ns/iter reference)
import jax
import jax.numpy as jnp
from jax import lax
from jax.experimental import pallas as pl
from jax.experimental.pallas import tpu as pltpu


def kernel(x):
    _, m, n2 = x.shape
    n = n2 // 2

    def body(x_hbm, out_ref, comm_ref, local_sem, send_sem, recv_sem):
        my_x = lax.axis_index("x")
        my_y = lax.axis_index("y")
        my_z = lax.axis_index("z")
        peer = (my_x, 1 - my_y, my_z)

        barrier_sem = pltpu.get_barrier_semaphore()
        pl.semaphore_signal(
            barrier_sem, inc=1, device_id=peer,
            device_id_type=pl.DeviceIdType.MESH,
        )
        pl.semaphore_wait(barrier_sem, 1)

        rdma = pltpu.make_async_remote_copy(
            src_ref=x_hbm.at[0, :, pl.ds((1 - my_y) * n, n)],
            dst_ref=comm_ref,
            send_sem=send_sem,
            recv_sem=recv_sem,
            device_id=peer,
            device_id_type=pl.DeviceIdType.MESH,
        )
        rdma.start()

        local = pltpu.make_async_copy(
            x_hbm.at[0, :, pl.ds(my_y * n, n)], out_ref, local_sem
        )
        local.start()
        local.wait()

        rdma.wait()
        out_ref[:, :] += comm_ref[:, :]

    return pl.pallas_call(
        body,
        out_shape=jax.ShapeDtypeStruct((m, n), jnp.float32),
        in_specs=[pl.BlockSpec(memory_space=pl.ANY)],
        out_specs=pl.BlockSpec(memory_space=pltpu.VMEM),
        scratch_shapes=[
            pltpu.VMEM((m, n), jnp.float32),
            pltpu.SemaphoreType.DMA,
            pltpu.SemaphoreType.DMA,
            pltpu.SemaphoreType.DMA,
        ],
        compiler_params=pltpu.CompilerParams(collective_id=0),
    )(x)


# device time: 108573 ns/iter; 1.7622x vs baseline; 1.7622x over previous
import jax
import jax.numpy as jnp
from jax import lax
from jax.experimental import pallas as pl
from jax.experimental.pallas import tpu as pltpu

C = 16


def kernel(x):
    _, m, n2 = x.shape
    n = n2 // 2
    h = n // 2
    rows = m // C

    def body(x_hbm, out_ref, f_ref, local_sem,
             y_send, y_recv, x_send, x_recv):
        my_x = lax.axis_index("x")
        my_y = lax.axis_index("y")
        my_z = lax.axis_index("z")
        y_peer = (my_x, 1 - my_y, my_z)
        x_peer = (1 - my_x, my_y, my_z)

        barrier_sem = pltpu.get_barrier_semaphore()
        for peer in (y_peer, x_peer):
            pl.semaphore_signal(
                barrier_sem, inc=1, device_id=peer,
                device_id_type=pl.DeviceIdType.MESH,
            )
        pl.semaphore_wait(barrier_sem, 2)

        local = pltpu.make_async_copy(
            x_hbm.at[0, :, pl.ds(my_y * n, n)], out_ref, local_sem
        )
        local.start()

        src_col = (1 - my_y) * n + my_x * h
        dst_col = my_x * h

        y_rdmas = []
        for k in range(C):
            r = pl.ds(k * rows, rows)
            rdma = pltpu.make_async_remote_copy(
                src_ref=x_hbm.at[0, r, pl.ds(src_col, h)],
                dst_ref=f_ref.at[r, pl.ds(dst_col, h)],
                send_sem=y_send.at[k],
                recv_sem=y_recv.at[k],
                device_id=y_peer,
                device_id_type=pl.DeviceIdType.MESH,
            )
            rdma.start()
            y_rdmas.append(rdma)

        local.wait()

        x_rdmas = []
        for k in range(C):
            r = pl.ds(k * rows, rows)
            y_rdmas[k].wait_recv()
            relay = pltpu.make_async_remote_copy(
                src_ref=f_ref.at[r, pl.ds(dst_col, h)],
                dst_ref=f_ref.at[r, pl.ds(dst_col, h)],
                send_sem=x_send.at[k],
                recv_sem=x_recv.at[k],
                device_id=x_peer,
                device_id_type=pl.DeviceIdType.MESH,
            )
            relay.start()
            x_rdmas.append(relay)
            if k > 0:
                x_rdmas[k - 1].wait_recv()
                rp = pl.ds((k - 1) * rows, rows)
                out_ref[rp, :] += f_ref[rp, :]

        x_rdmas[C - 1].wait_recv()
        rp = pl.ds((C - 1) * rows, rows)
        out_ref[rp, :] += f_ref[rp, :]

        for k in range(C):
            y_rdmas[k].wait_send()
            x_rdmas[k].wait_send()

    return pl.pallas_call(
        body,
        out_shape=jax.ShapeDtypeStruct((m, n), jnp.float32),
        in_specs=[pl.BlockSpec(memory_space=pl.ANY)],
        out_specs=pl.BlockSpec(memory_space=pltpu.VMEM),
        scratch_shapes=[
            pltpu.VMEM((m, n), jnp.float32),
            pltpu.SemaphoreType.DMA,
            pltpu.SemaphoreType.DMA((C,)),
            pltpu.SemaphoreType.DMA((C,)),
            pltpu.SemaphoreType.DMA((C,)),
            pltpu.SemaphoreType.DMA((C,)),
        ],
        compiler_params=pltpu.CompilerParams(collective_id=0),
    )(x)


# device time: 77165 ns/iter; 2.4795x vs baseline; 1.4070x over previous
import jax
import jax.numpy as jnp
from jax import lax
from jax.experimental import pallas as pl
from jax.experimental.pallas import tpu as pltpu

NC = 16
BAND_Y = range(0, 6)
BAND_X = range(6, 11)
BAND_Z = range(11, 16)
NY = len(BAND_Y)


def kernel(x):
    _, m, n2 = x.shape
    n = n2 // 2
    qw = n // 4
    rows = m // NC

    def body(x_hbm, out_ref, fq, local_sem,
             y_send, y_recv, xa_send, xa_recv, za_send, za_recv,
             xr_send, xr_recv, zr_send, zr_recv):
        my_x = lax.axis_index("x")
        my_y = lax.axis_index("y")
        my_z = lax.axis_index("z")
        zp = my_z % 2
        y_peer = (my_x, 1 - my_y, my_z)
        x_peer = (1 - my_x, my_y, my_z)
        z_peer = (my_x, my_y, my_z + 1 - 2 * zp)

        qc_me = (2 * zp + my_x) * qw
        qc_x = (2 * zp + (1 - my_x)) * qw
        qc_z = (2 * (1 - zp) + my_x) * qw
        qc_diag = (2 * (1 - zp) + (1 - my_x)) * qw

        def mk(src, dst, ssem, rsem, dev):
            return pltpu.make_async_remote_copy(
                src_ref=src, dst_ref=dst, send_sem=ssem, recv_sem=rsem,
                device_id=dev, device_id_type=pl.DeviceIdType.MESH,
            )

        local = pltpu.make_async_copy(
            x_hbm.at[0, :, pl.ds(my_y * n, n)], out_ref, local_sem
        )
        local.start()

        barrier_sem = pltpu.get_barrier_semaphore()
        for peer in (y_peer, x_peer, z_peer):
            pl.semaphore_signal(
                barrier_sem, inc=1, device_id=peer,
                device_id_type=pl.DeviceIdType.MESH,
            )
        pl.semaphore_wait(barrier_sem, 3)

        fcol = (1 - my_y) * n

        y_out = []
        for k in range(NC):
            r = pl.ds(k * rows, rows)
            rd = mk(x_hbm.at[0, r, pl.ds(fcol + qc_me, qw)],
                    fq.at[r, pl.ds(qc_me, qw)],
                    y_send.at[k], y_recv.at[k], y_peer)
            rd.start()
            y_out.append(rd)
        for j, k in enumerate(BAND_Y):
            r = pl.ds(k * rows, rows)
            rd = mk(x_hbm.at[0, r, pl.ds(fcol + qc_diag, qw)],
                    fq.at[r, pl.ds(qc_diag, qw)],
                    y_send.at[NC + j], y_recv.at[NC + j], y_peer)
            rd.start()
            y_out.append(rd)

        xa_out, za_out = [], []
        for k in range(NC):
            r = pl.ds(k * rows, rows)
            y_out[k].wait_recv()
            rd = mk(fq.at[r, pl.ds(qc_me, qw)], fq.at[r, pl.ds(qc_me, qw)],
                    xa_send.at[k], xa_recv.at[k], x_peer)
            rd.start()
            xa_out.append(rd)
            rd = mk(fq.at[r, pl.ds(qc_me, qw)], fq.at[r, pl.ds(qc_me, qw)],
                    za_send.at[k], za_recv.at[k], z_peer)
            rd.start()
            za_out.append(rd)

        xa_in = [mk(fq.at[pl.ds(k * rows, rows), pl.ds(qc_x, qw)],
                    fq.at[pl.ds(k * rows, rows), pl.ds(qc_x, qw)],
                    xa_send.at[k], xa_recv.at[k], x_peer)
                 for k in range(NC)]
        za_in = [mk(fq.at[pl.ds(k * rows, rows), pl.ds(qc_z, qw)],
                    fq.at[pl.ds(k * rows, rows), pl.ds(qc_z, qw)],
                    za_send.at[k], za_recv.at[k], z_peer)
                 for k in range(NC)]

        xr_out, zr_out = [], []
        for k in range(NC):
            r = pl.ds(k * rows, rows)
            xa_in[k].wait_recv()
            za_in[k].wait_recv()
            if k in BAND_X:
                j = k - BAND_X.start
                rd = mk(fq.at[r, pl.ds(qc_z, qw)], fq.at[r, pl.ds(qc_z, qw)],
                        xr_send.at[j], xr_recv.at[j], x_peer)
                rd.start()
                xr_out.append(rd)
            if k in BAND_Z:
                j = k - BAND_Z.start
                rd = mk(fq.at[r, pl.ds(qc_x, qw)], fq.at[r, pl.ds(qc_x, qw)],
                        zr_send.at[j], zr_recv.at[j], z_peer)
                rd.start()
                zr_out.append(rd)

        local.wait()
        for k in range(NC):
            r = pl.ds(k * rows, rows)
            if k in BAND_Y:
                y_out[NC + (k - BAND_Y.start)].wait_recv()
            elif k in BAND_X:
                j = k - BAND_X.start
                mk(fq.at[r, pl.ds(qc_diag, qw)], fq.at[r, pl.ds(qc_diag, qw)],
                   xr_send.at[j], xr_recv.at[j], x_peer).wait_recv()
            else:
                j = k - BAND_Z.start
                mk(fq.at[r, pl.ds(qc_diag, qw)], fq.at[r, pl.ds(qc_diag, qw)],
                   zr_send.at[j], zr_recv.at[j], z_peer).wait_recv()
            out_ref[r, :] += fq[r, :]

        for rd in y_out + xa_out + za_out + xr_out + zr_out:
            rd.wait_send()

    return pl.pallas_call(
        body,
        out_shape=jax.ShapeDtypeStruct((m, n), jnp.float32),
        in_specs=[pl.BlockSpec(memory_space=pl.ANY)],
        out_specs=pl.BlockSpec(memory_space=pltpu.VMEM),
        scratch_shapes=[
            pltpu.VMEM((m, n), jnp.float32),
            pltpu.SemaphoreType.DMA,
            pltpu.SemaphoreType.DMA((NC + NY,)),
            pltpu.SemaphoreType.DMA((NC + NY,)),
            pltpu.SemaphoreType.DMA((NC,)),
            pltpu.SemaphoreType.DMA((NC,)),
            pltpu.SemaphoreType.DMA((NC,)),
            pltpu.SemaphoreType.DMA((NC,)),
            pltpu.SemaphoreType.DMA((len(BAND_X),)),
            pltpu.SemaphoreType.DMA((len(BAND_X),)),
            pltpu.SemaphoreType.DMA((len(BAND_Z),)),
            pltpu.SemaphoreType.DMA((len(BAND_Z),)),
        ],
        compiler_params=pltpu.CompilerParams(collective_id=0),
    )(x)
